# baseline (speedup 1.0000x reference)
import functools

import jax
import jax.numpy as jnp
from jax.experimental import pallas as pl
from jax.experimental.pallas import tpu as pltpu

EPS_GUARD = 0.0001


def _ffn_body(x_ref, m_ref, w1_ref, b1_ref, w2_ref, b2_ref, o_ref, acc_ref,
              *, n_experts, n_ftiles):
    e = pl.program_id(1)
    f = pl.program_id(2)

    @pl.when((e == 0) & (f == 0))
    def _init():
        acc_ref[...] = jnp.zeros_like(acc_ref)

    h = jnp.dot(x_ref[...], w1_ref[0], preferred_element_type=jnp.float32)
    h = jnp.maximum(h + b1_ref[0], 0.0)
    part = jnp.dot(h, w2_ref[0], preferred_element_type=jnp.float32)

    @pl.when(f == 0)
    def _bias2():
        acc_ref[...] += b2_ref[0] * m_ref[0]

    acc_ref[...] += part * m_ref[0]

    @pl.when((e == n_experts - 1) & (f == n_ftiles - 1))
    def _flush():
        o_ref[...] = acc_ref[...]


def kernel(x, presence, W1, b1, W2, b2):
    N, D = x.shape
    E, _, F = W1.shape

    TN = min(512, N)
    TF = min(1024, F)
    n_itiles = N // TN
    n_ftiles = F // TF

    mask = (presence.T > EPS_GUARD).astype(jnp.float32)[:, :, None]
    b1r = b1[:, None, :]
    b2r = b2[:, None, :]

    body = functools.partial(_ffn_body, n_experts=E, n_ftiles=n_ftiles)

    out = pl.pallas_call(
        body,
        grid=(n_itiles, E, n_ftiles),
        in_specs=[
            pl.BlockSpec((TN, D), lambda i, e, f: (i, 0)),
            pl.BlockSpec((1, TN, 1), lambda i, e, f: (e, i, 0)),
            pl.BlockSpec((1, D, TF), lambda i, e, f: (e, 0, f)),
            pl.BlockSpec((1, 1, TF), lambda i, e, f: (e, 0, f)),
            pl.BlockSpec((1, TF, D), lambda i, e, f: (e, f, 0)),
            pl.BlockSpec((1, 1, D), lambda i, e, f: (e, 0, 0)),
        ],
        out_specs=pl.BlockSpec((TN, D), lambda i, e, f: (i, 0)),
        out_shape=jax.ShapeDtypeStruct((N, D), jnp.float32),
        scratch_shapes=[pltpu.VMEM((TN, D), jnp.float32)],
        compiler_params=pltpu.CompilerParams(
            dimension_semantics=("parallel", "arbitrary", "arbitrary"),
        ),
    )(x, mask, W1, b1r, W2, b2r)
    return out
